# Initial kernel scaffold; baseline (speedup 1.0000x reference)
import jax
import jax.numpy as jnp
from jax.experimental import pallas as pl


def kernel(hn, he, edge_index, W, b):
    raise NotImplementedError("write your pallas kernel here")


# speedup vs baseline: 6.6031x; 6.6031x over previous
import functools

import jax
import jax.numpy as jnp
from jax import lax
from jax.experimental import pallas as pl
from jax.experimental.pallas import tpu as pltpu
from jax.experimental.pallas import tpu_sc as plsc

IN_F = 128
N_NODES = 10000
N_EDGES = 320000

_R_BLK = 2560
_PQ_BLK = 2000

_NC = 2
_NS = 16
_NW = _NC * _NS
_CHUNK = N_EDGES // _NW
_L = 16


def _r_body(he_ref, w3_ref, b_ref, out_ref):
    out_ref[...] = (
        jnp.dot(he_ref[...], w3_ref[...], preferred_element_type=jnp.float32)
        + b_ref[0, 0]
    )


def _pq_body(hn_ref, wpq_ref, p_ref, q_ref):
    res = jnp.dot(hn_ref[...], wpq_ref[...], preferred_element_type=jnp.float32)
    p_ref[...] = res[:, 0:1]
    q_ref[...] = res[:, 1:2]


def _sc_body(p_hbm, q_hbm, r_hbm, src_hbm, dst_hbm, out_hbm,
             p_v, q_v, src_v, dst_v, r_v, out_v):
    wid = lax.axis_index("s") * _NC + lax.axis_index("c")
    base = wid * _CHUNK

    pltpu.sync_copy(p_hbm, p_v)
    pltpu.sync_copy(q_hbm, q_v)
    pltpu.sync_copy(src_hbm.at[pl.ds(base, _CHUNK)], src_v)
    pltpu.sync_copy(dst_hbm.at[pl.ds(base, _CHUNK)], dst_v)
    pltpu.sync_copy(r_hbm.at[pl.ds(base, _CHUNK)], r_v)

    def body(i, carry):
        s = i * _L
        si = src_v[pl.ds(s, _L)]
        di = dst_v[pl.ds(s, _L)]
        pv = plsc.load_gather(p_v, [si])
        qv = plsc.load_gather(q_v, [di])
        out_v[pl.ds(s, _L)] = pv + qv + r_v[pl.ds(s, _L)]
        return carry

    lax.fori_loop(0, _CHUNK // _L, body, 0)

    pltpu.sync_copy(out_v, out_hbm.at[pl.ds(base, _CHUNK)])


def kernel(hn, he, edge_index, W, b):
    edge_index = edge_index.astype(jnp.int32)
    src = edge_index[0]
    dst = edge_index[1]

    w3 = W[0, 2 * IN_F:].reshape(IN_F, 1)
    wpq = jnp.stack([W[0, :IN_F], W[0, IN_F:2 * IN_F]], axis=1)
    b2 = b.reshape(1, 1)

    r = pl.pallas_call(
        _r_body,
        grid=(N_EDGES // _R_BLK,),
        in_specs=[
            pl.BlockSpec((_R_BLK, IN_F), lambda i: (i, 0)),
            pl.BlockSpec((IN_F, 1), lambda i: (0, 0)),
            pl.BlockSpec((1, 1), lambda i: (0, 0)),
        ],
        out_specs=pl.BlockSpec((_R_BLK, 1), lambda i: (i, 0)),
        out_shape=jax.ShapeDtypeStruct((N_EDGES, 1), jnp.float32),
    )(he, w3, b2)

    p, q = pl.pallas_call(
        _pq_body,
        grid=(N_NODES // _PQ_BLK,),
        in_specs=[
            pl.BlockSpec((_PQ_BLK, IN_F), lambda i: (i, 0)),
            pl.BlockSpec((IN_F, 2), lambda i: (0, 0)),
        ],
        out_specs=[
            pl.BlockSpec((_PQ_BLK, 1), lambda i: (i, 0)),
            pl.BlockSpec((_PQ_BLK, 1), lambda i: (i, 0)),
        ],
        out_shape=[
            jax.ShapeDtypeStruct((N_NODES, 1), jnp.float32),
            jax.ShapeDtypeStruct((N_NODES, 1), jnp.float32),
        ],
    )(hn, wpq)

    sc = functools.partial(
        pl.kernel,
        mesh=plsc.VectorSubcoreMesh(core_axis_name="c", subcore_axis_name="s"),
        compiler_params=pltpu.CompilerParams(needs_layout_passes=False),
        out_type=jax.ShapeDtypeStruct((N_EDGES,), jnp.float32),
        scratch_types=[
            pltpu.VMEM((N_NODES,), jnp.float32),
            pltpu.VMEM((N_NODES,), jnp.float32),
            pltpu.VMEM((_CHUNK,), jnp.int32),
            pltpu.VMEM((_CHUNK,), jnp.int32),
            pltpu.VMEM((_CHUNK,), jnp.float32),
            pltpu.VMEM((_CHUNK,), jnp.float32),
        ],
    )(_sc_body)

    score = sc(p.reshape(N_NODES), q.reshape(N_NODES), r.reshape(N_EDGES),
               src, dst)
    return score.reshape(N_EDGES, 1)
